# baseline (speedup 1.0000x reference)
import functools

import jax
import jax.numpy as jnp
from jax import lax
from jax.experimental import pallas as pl
from jax.experimental.pallas import tpu as pltpu
from jax.experimental.pallas import tpu_sc as plsc

N = 100000
E = 800000
H = 128
W8 = 8
FEATS = {'v': 7, 'e': 2, 'f': 5}
REL_LIST = [('v', 'v'), ('v', 'e'), ('v', 'f'), ('e', 'v'), ('e', 'f'),
            ('f', 'v'), ('f', 'e')]
SRC_GROUPS = [('v', [0, 1, 2]), ('e', [3, 4]), ('f', [5, 6])]
DST_RELS = {'v': ['vv', 'ev', 'fv'], 'e': ['ve', 'fe'], 'f': ['vf', 'ef']}
RIDX = {s + d: i for i, (s, d) in enumerate(REL_LIST)}
GOFF = {'v': 0, 'e': 8, 'f': 16}

NC, NS = 2, 16
NW = NC * NS
SUB = 200
NSUB = 5
CH = SUB * NSUB
NCH = 25
EPW = CH * NCH
FPT = 6256
NP = NS * FPT


def _sc_segment_sums(x_all, eidx, zeros_hbm):
    mesh = plsc.VectorSubcoreMesh(core_axis_name="c", subcore_axis_name="s")

    @functools.partial(
        pl.kernel,
        out_type=jax.ShapeDtypeStruct((NP, 128), jnp.float32),
        mesh=mesh,
        compiler_params=pltpu.CompilerParams(use_tc_tiling_on_sc=False),
        scratch_types=[
            pltpu.VMEM((CH,), jnp.int32),
            pltpu.VMEM((CH,), jnp.int32),
            pltpu.VMEM((CH,), jnp.int32),
            pltpu.VMEM((CH,), jnp.int32),
            pltpu.VMEM((CH, W8), jnp.float32),
            pltpu.VMEM((CH, W8), jnp.float32),
            pltpu.VMEM_SHARED((NP, W8), jnp.float32),
            pltpu.VMEM_SHARED((NP, W8), jnp.float32),
            pltpu.SemaphoreType.DMA,
            pltpu.SemaphoreType.DMA,
            pltpu.SemaphoreType.DMA,
            pltpu.SemaphoreType.DMA,
        ],
    )
    def body(xa_ref, *refs):
        idx_refs = refs[:14]
        z_ref = refs[14]
        out_ref = refs[15]
        (sidxA, didxA, sidxB, didxB, rowsA, rowsB, table, acc,
         gsemA, gsemB, ssemA, ssemB) = refs[16:]
        cid = lax.axis_index("c")
        sid = lax.axis_index("s")
        wid = sid * NC + cid
        row0 = sid * FPT
        for src_t, rels in SRC_GROUPS:
            pltpu.sync_copy(
                xa_ref.at[pl.ds(row0, FPT), pl.ds(GOFF[src_t], W8)],
                table.at[pl.ds(row0, FPT)])
            for r in rels:
                si_ref = idx_refs[2 * r]
                di_ref = idx_refs[2 * r + 1]
                pltpu.sync_copy(z_ref, acc.at[pl.ds(sid * FPT, FPT)])
                plsc.subcore_barrier()
                e0 = wid * EPW

                def load_idx(si_buf, di_buf, c):
                    eb = e0 + c * CH
                    pltpu.sync_copy(si_ref.at[pl.ds(eb, CH)], si_buf)
                    pltpu.sync_copy(di_ref.at[pl.ds(eb, CH)], di_buf)

                def fire_gathers(si_buf, rows_buf, sem):
                    for j in range(NSUB):
                        pltpu.async_copy(
                            table.at[si_buf.at[pl.ds(j * SUB, SUB)]],
                            rows_buf.at[pl.ds(j * SUB, SUB)], sem)

                def wait_gathers(si_buf, rows_buf, sem):
                    for j in range(NSUB):
                        pltpu.make_async_copy(
                            table.at[si_buf.at[pl.ds(j * SUB, SUB)]],
                            rows_buf.at[pl.ds(j * SUB, SUB)], sem).wait()

                def fire_scatters(di_buf, rows_buf, sem):
                    for j in range(NSUB):
                        pltpu.async_copy(
                            rows_buf.at[pl.ds(j * SUB, SUB)],
                            acc.at[di_buf.at[pl.ds(j * SUB, SUB)]], sem,
                            add=True)

                def wait_scatters(di_buf, rows_buf, sem):
                    for j in range(NSUB):
                        pltpu.make_async_copy(
                            rows_buf.at[pl.ds(j * SUB, SUB)],
                            acc.at[di_buf.at[pl.ds(j * SUB, SUB)]],
                            sem).wait()

                load_idx(sidxA, didxA, 0)
                fire_gathers(sidxA, rowsA, gsemA)

                def pair(i, carry):
                    load_idx(sidxB, didxB, 2 * i + 1)
                    wait_gathers(sidxA, rowsA, gsemA)
                    fire_scatters(didxA, rowsA, ssemA)
                    fire_gathers(sidxB, rowsB, gsemB)
                    wait_scatters(didxA, rowsA, ssemA)
                    load_idx(sidxA, didxA, 2 * i + 2)
                    fire_gathers(sidxA, rowsA, gsemA)
                    wait_gathers(sidxB, rowsB, gsemB)
                    fire_scatters(didxB, rowsB, ssemB)
                    wait_scatters(didxB, rowsB, ssemB)
                    return carry

                lax.fori_loop(0, (NCH - 1) // 2, pair, 0)
                wait_gathers(sidxA, rowsA, gsemA)
                fire_scatters(didxA, rowsA, ssemA)
                wait_scatters(didxA, rowsA, ssemA)
                plsc.subcore_barrier()
                pltpu.sync_copy(
                    acc.at[pl.ds(sid * FPT, FPT)],
                    out_ref.at[pl.ds(sid * FPT, FPT),
                               pl.ds((2 * r + cid) * W8, W8)])
                plsc.subcore_barrier()

    return body(x_all, *eidx, zeros_hbm)


def _tc_epilogue(x_all, slab, weights, blk=2000):
    def tc_body(x_ref, s_ref, w_ref, ov_ref, oe_ref, of_ref):
        xa = x_ref[...]
        s = s_ref[...]
        o_refs = {'v': ov_ref, 'e': oe_ref, 'f': of_ref}
        wrow = 0
        for d in 'vef':
            x = xa[:, GOFF[d]:GOFF[d] + W8]
            out = jnp.dot(x, w_ref[wrow:wrow + W8, :],
                          preferred_element_type=jnp.float32)
            wrow += W8
            for rname in DST_RELS[d]:
                r = RIDX[rname]
                fs = FEATS[rname[0]]
                a = (s[:, (2 * r) * W8:(2 * r + 1) * W8]
                     + s[:, (2 * r + 1) * W8:(2 * r + 2) * W8])
                cnt = a[:, fs:fs + 1]
                out += jnp.dot(a / jnp.maximum(cnt, 1.0),
                               w_ref[wrow:wrow + W8, :],
                               preferred_element_type=jnp.float32)
                wrow += W8
            o_refs[d][...] = jnp.maximum(out, 0.0)

    nw_rows = W8 * (3 + 7)
    return pl.pallas_call(
        tc_body,
        grid=(N // blk,),
        in_specs=[
            pl.BlockSpec((blk, 24), lambda i: (i, 0)),
            pl.BlockSpec((blk, 128), lambda i: (i, 0)),
            pl.BlockSpec((nw_rows, H), lambda i: (0, 0)),
        ],
        out_specs=[pl.BlockSpec((blk, H), lambda i: (i, 0))] * 3,
        out_shape=[jax.ShapeDtypeStruct((N, H), jnp.float32)] * 3,
    )(x_all, slab, weights)


def kernel(x_v, x_e, x_f, params, ei_vv, ei_ve, ei_vf, ei_ev, ei_ef,
           ei_fv, ei_fe):
    eis = {'vv': ei_vv, 've': ei_ve, 'vf': ei_vf, 'ev': ei_ev,
           'ef': ei_ef, 'fv': ei_fv, 'fe': ei_fe}

    one = jnp.ones((N, 1), jnp.float32)
    zero = jnp.zeros((N, 1), jnp.float32)
    x_cat = jnp.concatenate(
        [x_v, one, x_e, one, zero, zero, zero, zero, zero,
         x_f, one, zero, zero], axis=1)
    x_all = jnp.zeros((NP, 24), jnp.float32).at[:N].set(x_cat)

    eidx = []
    for s, d in REL_LIST:
        ei = eis[s + d]
        eidx.append(ei[0])
        eidx.append(ei[1])
    zeros_hbm = jnp.zeros((FPT, W8), jnp.float32)

    slab = _sc_segment_sums(x_all, eidx, zeros_hbm)

    wmats = []
    for d in 'vef':
        rels = DST_RELS[d]
        K = float(len(rels))
        Fd = FEATS[d]
        wmats.append(jnp.zeros((W8, H), jnp.float32)
                     .at[:Fd, :].set(params['Ws_' + d]
                                     + sum(params['Wr_' + r]
                                           for r in rels) / K)
                     .at[Fd, :].set(params['bs_' + d]
                                    + sum(params['bl_' + r]
                                          + params['br_' + r]
                                          for r in rels) / K))
        for r in rels:
            wmats.append(jnp.zeros((W8, H), jnp.float32)
                         .at[:FEATS[r[0]], :].set(params['Wl_' + r] / K))
    weights = jnp.concatenate(wmats, axis=0)

    out_v, out_e, out_f = _tc_epilogue(x_all, slab, weights)
    return (out_v, out_e, out_f)


# speedup vs baseline: 1.0132x; 1.0132x over previous
import functools

import jax
import jax.numpy as jnp
from jax import lax
from jax.experimental import pallas as pl
from jax.experimental.pallas import tpu as pltpu
from jax.experimental.pallas import tpu_sc as plsc

N = 100000
E = 800000
H = 128
W8 = 8
FEATS = {'v': 7, 'e': 2, 'f': 5}
REL_LIST = [('v', 'v'), ('v', 'e'), ('v', 'f'), ('e', 'v'), ('e', 'f'),
            ('f', 'v'), ('f', 'e')]
SRC_GROUPS = [('v', [0, 1, 2]), ('e', [3, 4]), ('f', [5, 6])]
DST_RELS = {'v': ['vv', 'ev', 'fv'], 'e': ['ve', 'fe'], 'f': ['vf', 'ef']}
RIDX = {s + d: i for i, (s, d) in enumerate(REL_LIST)}
GOFF = {'v': 0, 'e': 8, 'f': 16}

NC, NS = 2, 16
NW = NC * NS
SUB = 1000
NSUB = 1
CH = SUB * NSUB
NCH = 25
EPW = CH * NCH
FPT = 6256
NP = NS * FPT


def _sc_segment_sums(x_all, eidx, zeros_hbm):
    mesh = plsc.VectorSubcoreMesh(core_axis_name="c", subcore_axis_name="s")

    @functools.partial(
        pl.kernel,
        out_type=jax.ShapeDtypeStruct((NP, 128), jnp.float32),
        mesh=mesh,
        compiler_params=pltpu.CompilerParams(use_tc_tiling_on_sc=False),
        scratch_types=[
            pltpu.VMEM((CH,), jnp.int32),
            pltpu.VMEM((CH,), jnp.int32),
            pltpu.VMEM((CH,), jnp.int32),
            pltpu.VMEM((CH,), jnp.int32),
            pltpu.VMEM((CH, W8), jnp.float32),
            pltpu.VMEM((CH, W8), jnp.float32),
            pltpu.VMEM_SHARED((NP, W8), jnp.float32),
            pltpu.VMEM_SHARED((NP, W8), jnp.float32),
            pltpu.SemaphoreType.DMA,
            pltpu.SemaphoreType.DMA,
            pltpu.SemaphoreType.DMA,
            pltpu.SemaphoreType.DMA,
        ],
    )
    def body(xa_ref, *refs):
        idx_refs = refs[:14]
        z_ref = refs[14]
        out_ref = refs[15]
        (sidxA, didxA, sidxB, didxB, rowsA, rowsB, table, acc,
         gsemA, gsemB, ssemA, ssemB) = refs[16:]
        cid = lax.axis_index("c")
        sid = lax.axis_index("s")
        wid = sid * NC + cid
        row0 = sid * FPT
        for src_t, rels in SRC_GROUPS:
            pltpu.sync_copy(
                xa_ref.at[pl.ds(row0, FPT), pl.ds(GOFF[src_t], W8)],
                table.at[pl.ds(row0, FPT)])
            for r in rels:
                si_ref = idx_refs[2 * r]
                di_ref = idx_refs[2 * r + 1]
                pltpu.sync_copy(z_ref, acc.at[pl.ds(sid * FPT, FPT)])
                plsc.subcore_barrier()
                e0 = wid * EPW

                def load_idx(si_buf, di_buf, c):
                    eb = e0 + c * CH
                    pltpu.sync_copy(si_ref.at[pl.ds(eb, CH)], si_buf)
                    pltpu.sync_copy(di_ref.at[pl.ds(eb, CH)], di_buf)

                def fire_gathers(si_buf, rows_buf, sem):
                    for j in range(NSUB):
                        pltpu.async_copy(
                            table.at[si_buf.at[pl.ds(j * SUB, SUB)]],
                            rows_buf.at[pl.ds(j * SUB, SUB)], sem)

                def wait_gathers(si_buf, rows_buf, sem):
                    for j in range(NSUB):
                        pltpu.make_async_copy(
                            table.at[si_buf.at[pl.ds(j * SUB, SUB)]],
                            rows_buf.at[pl.ds(j * SUB, SUB)], sem).wait()

                def fire_scatters(di_buf, rows_buf, sem):
                    for j in range(NSUB):
                        pltpu.async_copy(
                            rows_buf.at[pl.ds(j * SUB, SUB)],
                            acc.at[di_buf.at[pl.ds(j * SUB, SUB)]], sem,
                            add=True)

                def wait_scatters(di_buf, rows_buf, sem):
                    for j in range(NSUB):
                        pltpu.make_async_copy(
                            rows_buf.at[pl.ds(j * SUB, SUB)],
                            acc.at[di_buf.at[pl.ds(j * SUB, SUB)]],
                            sem).wait()

                load_idx(sidxA, didxA, 0)
                fire_gathers(sidxA, rowsA, gsemA)

                def pair(i, carry):
                    load_idx(sidxB, didxB, 2 * i + 1)
                    wait_gathers(sidxA, rowsA, gsemA)
                    fire_scatters(didxA, rowsA, ssemA)
                    fire_gathers(sidxB, rowsB, gsemB)
                    wait_scatters(didxA, rowsA, ssemA)
                    load_idx(sidxA, didxA, 2 * i + 2)
                    fire_gathers(sidxA, rowsA, gsemA)
                    wait_gathers(sidxB, rowsB, gsemB)
                    fire_scatters(didxB, rowsB, ssemB)
                    wait_scatters(didxB, rowsB, ssemB)
                    return carry

                lax.fori_loop(0, (NCH - 1) // 2, pair, 0)
                wait_gathers(sidxA, rowsA, gsemA)
                fire_scatters(didxA, rowsA, ssemA)
                wait_scatters(didxA, rowsA, ssemA)
                plsc.subcore_barrier()
                pltpu.sync_copy(
                    acc.at[pl.ds(sid * FPT, FPT)],
                    out_ref.at[pl.ds(sid * FPT, FPT),
                               pl.ds((2 * r + cid) * W8, W8)])
                plsc.subcore_barrier()

    return body(x_all, *eidx, zeros_hbm)


def _tc_epilogue(x_all, slab, weights, blk=4000):
    def tc_body(x_ref, s_ref, w_ref, ov_ref, oe_ref, of_ref):
        xa = x_ref[...]
        s = s_ref[...]
        o_refs = {'v': ov_ref, 'e': oe_ref, 'f': of_ref}
        wrow = 0
        for d in 'vef':
            x = xa[:, GOFF[d]:GOFF[d] + W8]
            out = jnp.dot(x, w_ref[wrow:wrow + W8, :],
                          preferred_element_type=jnp.float32)
            wrow += W8
            for rname in DST_RELS[d]:
                r = RIDX[rname]
                fs = FEATS[rname[0]]
                a = (s[:, (2 * r) * W8:(2 * r + 1) * W8]
                     + s[:, (2 * r + 1) * W8:(2 * r + 2) * W8])
                cnt = a[:, fs:fs + 1]
                out += jnp.dot(a / jnp.maximum(cnt, 1.0),
                               w_ref[wrow:wrow + W8, :],
                               preferred_element_type=jnp.float32)
                wrow += W8
            o_refs[d][...] = jnp.maximum(out, 0.0)

    nw_rows = W8 * (3 + 7)
    return pl.pallas_call(
        tc_body,
        grid=(N // blk,),
        in_specs=[
            pl.BlockSpec((blk, 24), lambda i: (i, 0)),
            pl.BlockSpec((blk, 128), lambda i: (i, 0)),
            pl.BlockSpec((nw_rows, H), lambda i: (0, 0)),
        ],
        out_specs=[pl.BlockSpec((blk, H), lambda i: (i, 0))] * 3,
        out_shape=[jax.ShapeDtypeStruct((N, H), jnp.float32)] * 3,
    )(x_all, slab, weights)


def kernel(x_v, x_e, x_f, params, ei_vv, ei_ve, ei_vf, ei_ev, ei_ef,
           ei_fv, ei_fe):
    eis = {'vv': ei_vv, 've': ei_ve, 'vf': ei_vf, 'ev': ei_ev,
           'ef': ei_ef, 'fv': ei_fv, 'fe': ei_fe}

    one = jnp.ones((N, 1), jnp.float32)
    zero = jnp.zeros((N, 1), jnp.float32)
    x_cat = jnp.concatenate(
        [x_v, one, x_e, one, zero, zero, zero, zero, zero,
         x_f, one, zero, zero], axis=1)
    x_all = jnp.zeros((NP, 24), jnp.float32).at[:N].set(x_cat)

    eidx = []
    for s, d in REL_LIST:
        ei = eis[s + d]
        eidx.append(ei[0])
        eidx.append(ei[1])
    zeros_hbm = jnp.zeros((FPT, W8), jnp.float32)

    slab = _sc_segment_sums(x_all, eidx, zeros_hbm)

    wmats = []
    for d in 'vef':
        rels = DST_RELS[d]
        K = float(len(rels))
        Fd = FEATS[d]
        wmats.append(jnp.zeros((W8, H), jnp.float32)
                     .at[:Fd, :].set(params['Ws_' + d]
                                     + sum(params['Wr_' + r]
                                           for r in rels) / K)
                     .at[Fd, :].set(params['bs_' + d]
                                    + sum(params['bl_' + r]
                                          + params['br_' + r]
                                          for r in rels) / K))
        for r in rels:
            wmats.append(jnp.zeros((W8, H), jnp.float32)
                         .at[:FEATS[r[0]], :].set(params['Wl_' + r] / K))
    weights = jnp.concatenate(wmats, axis=0)

    out_v, out_e, out_f = _tc_epilogue(x_all, slab, weights)
    return (out_v, out_e, out_f)


# speedup vs baseline: 1.0751x; 1.0611x over previous
import functools

import jax
import jax.numpy as jnp
from jax import lax
from jax.experimental import pallas as pl
from jax.experimental.pallas import tpu as pltpu
from jax.experimental.pallas import tpu_sc as plsc

N = 100000
E = 800000
H = 128
W8 = 8
FEATS = {'v': 7, 'e': 2, 'f': 5}
REL_LIST = [('v', 'v'), ('v', 'e'), ('v', 'f'), ('e', 'v'), ('e', 'f'),
            ('f', 'v'), ('f', 'e')]
SRC_GROUPS = [('v', [0, 1, 2]), ('e', [3, 4]), ('f', [5, 6])]
DST_RELS = {'v': ['vv', 'ev', 'fv'], 'e': ['ve', 'fe'], 'f': ['vf', 'ef']}
RIDX = {s + d: i for i, (s, d) in enumerate(REL_LIST)}
GOFF = {'v': 0, 'e': 8, 'f': 16}

NC, NS = 2, 16
NW = NC * NS
SUB = 1000
NSUB = 1
CH = SUB * NSUB
NCH = 25
EPW = CH * NCH
FPT = 6256
NP = NS * FPT


def _sc_segment_sums(x_all, eidx, zeros_hbm):
    mesh = plsc.VectorSubcoreMesh(core_axis_name="c", subcore_axis_name="s")

    @functools.partial(
        pl.kernel,
        out_type=jax.ShapeDtypeStruct((NP, 128), jnp.float32),
        mesh=mesh,
        compiler_params=pltpu.CompilerParams(use_tc_tiling_on_sc=False),
        scratch_types=[
            pltpu.VMEM((CH,), jnp.int32),
            pltpu.VMEM((CH,), jnp.int32),
            pltpu.VMEM((CH,), jnp.int32),
            pltpu.VMEM((CH,), jnp.int32),
            pltpu.VMEM((CH, W8), jnp.float32),
            pltpu.VMEM((CH, W8), jnp.float32),
            pltpu.VMEM_SHARED((NP, W8), jnp.float32),
            pltpu.VMEM_SHARED((NP, W8), jnp.float32),
            pltpu.SemaphoreType.DMA,
            pltpu.SemaphoreType.DMA,
            pltpu.SemaphoreType.DMA,
            pltpu.SemaphoreType.DMA,
        ],
    )
    def body(xa_ref, *refs):
        idx_refs = refs[:14]
        z_ref = refs[14]
        out_ref = refs[15]
        (sidxA, didxA, sidxB, didxB, rowsA, rowsB, table, acc,
         gsemA, gsemB, ssemA, ssemB) = refs[16:]
        cid = lax.axis_index("c")
        sid = lax.axis_index("s")
        wid = sid * NC + cid
        row0 = sid * FPT
        pltpu.sync_copy(z_ref, acc.at[pl.ds(sid * FPT, FPT)])
        for src_t, rels in SRC_GROUPS:
            pltpu.sync_copy(
                xa_ref.at[pl.ds(row0, FPT), pl.ds(GOFF[src_t], W8)],
                table.at[pl.ds(row0, FPT)])
            for r in rels:
                si_ref = idx_refs[2 * r]
                di_ref = idx_refs[2 * r + 1]
                plsc.subcore_barrier()
                e0 = wid * EPW

                def load_idx(si_buf, di_buf, c):
                    eb = e0 + c * CH
                    pltpu.sync_copy(si_ref.at[pl.ds(eb, CH)], si_buf)
                    pltpu.sync_copy(di_ref.at[pl.ds(eb, CH)], di_buf)

                def fire_gathers(si_buf, rows_buf, sem):
                    for j in range(NSUB):
                        pltpu.async_copy(
                            table.at[si_buf.at[pl.ds(j * SUB, SUB)]],
                            rows_buf.at[pl.ds(j * SUB, SUB)], sem)

                def wait_gathers(si_buf, rows_buf, sem):
                    for j in range(NSUB):
                        pltpu.make_async_copy(
                            table.at[si_buf.at[pl.ds(j * SUB, SUB)]],
                            rows_buf.at[pl.ds(j * SUB, SUB)], sem).wait()

                def fire_scatters(di_buf, rows_buf, sem):
                    for j in range(NSUB):
                        pltpu.async_copy(
                            rows_buf.at[pl.ds(j * SUB, SUB)],
                            acc.at[di_buf.at[pl.ds(j * SUB, SUB)]], sem,
                            add=True)

                def wait_scatters(di_buf, rows_buf, sem):
                    for j in range(NSUB):
                        pltpu.make_async_copy(
                            rows_buf.at[pl.ds(j * SUB, SUB)],
                            acc.at[di_buf.at[pl.ds(j * SUB, SUB)]],
                            sem).wait()

                load_idx(sidxA, didxA, 0)
                fire_gathers(sidxA, rowsA, gsemA)

                def pair(i, carry):
                    load_idx(sidxB, didxB, 2 * i + 1)
                    wait_gathers(sidxA, rowsA, gsemA)
                    fire_scatters(didxA, rowsA, ssemA)
                    fire_gathers(sidxB, rowsB, gsemB)
                    wait_scatters(didxA, rowsA, ssemA)
                    load_idx(sidxA, didxA, 2 * i + 2)
                    fire_gathers(sidxA, rowsA, gsemA)
                    wait_gathers(sidxB, rowsB, gsemB)
                    fire_scatters(didxB, rowsB, ssemB)
                    wait_scatters(didxB, rowsB, ssemB)
                    return carry

                lax.fori_loop(0, (NCH - 1) // 2, pair, 0)
                wait_gathers(sidxA, rowsA, gsemA)
                fire_scatters(didxA, rowsA, ssemA)
                wait_scatters(didxA, rowsA, ssemA)
                plsc.subcore_barrier()
                pltpu.sync_copy(
                    acc.at[pl.ds(sid * FPT, FPT)],
                    out_ref.at[pl.ds(sid * FPT, FPT),
                               pl.ds((2 * r + cid) * W8, W8)])
                pltpu.sync_copy(z_ref, acc.at[pl.ds(sid * FPT, FPT)])

    return body(x_all, *eidx, zeros_hbm)


def _tc_epilogue(x_all, slab, weights, blk=4000):
    def tc_body(x_ref, s_ref, w_ref, ov_ref, oe_ref, of_ref):
        xa = x_ref[...]
        s = s_ref[...]
        o_refs = {'v': ov_ref, 'e': oe_ref, 'f': of_ref}
        wrow = 0
        for d in 'vef':
            x = xa[:, GOFF[d]:GOFF[d] + W8]
            out = jnp.dot(x, w_ref[wrow:wrow + W8, :],
                          preferred_element_type=jnp.float32)
            wrow += W8
            for rname in DST_RELS[d]:
                r = RIDX[rname]
                fs = FEATS[rname[0]]
                a = (s[:, (2 * r) * W8:(2 * r + 1) * W8]
                     + s[:, (2 * r + 1) * W8:(2 * r + 2) * W8])
                cnt = a[:, fs:fs + 1]
                out += jnp.dot(a / jnp.maximum(cnt, 1.0),
                               w_ref[wrow:wrow + W8, :],
                               preferred_element_type=jnp.float32)
                wrow += W8
            o_refs[d][...] = jnp.maximum(out, 0.0)

    nw_rows = W8 * (3 + 7)
    return pl.pallas_call(
        tc_body,
        grid=(N // blk,),
        in_specs=[
            pl.BlockSpec((blk, 24), lambda i: (i, 0)),
            pl.BlockSpec((blk, 128), lambda i: (i, 0)),
            pl.BlockSpec((nw_rows, H), lambda i: (0, 0)),
        ],
        out_specs=[pl.BlockSpec((blk, H), lambda i: (i, 0))] * 3,
        out_shape=[jax.ShapeDtypeStruct((N, H), jnp.float32)] * 3,
    )(x_all, slab, weights)


def kernel(x_v, x_e, x_f, params, ei_vv, ei_ve, ei_vf, ei_ev, ei_ef,
           ei_fv, ei_fe):
    eis = {'vv': ei_vv, 've': ei_ve, 'vf': ei_vf, 'ev': ei_ev,
           'ef': ei_ef, 'fv': ei_fv, 'fe': ei_fe}

    one = jnp.ones((N, 1), jnp.float32)
    zero = jnp.zeros((N, 1), jnp.float32)
    x_cat = jnp.concatenate(
        [x_v, one, x_e, one, zero, zero, zero, zero, zero,
         x_f, one, zero, zero], axis=1)
    x_all = lax.pad(x_cat, jnp.float32(0),
                    [(0, NP - N, 0), (0, 0, 0)])

    eidx = []
    for s, d in REL_LIST:
        ei = eis[s + d]
        eidx.append(ei[0])
        eidx.append(ei[1])
    zeros_hbm = jnp.zeros((FPT, W8), jnp.float32)

    slab = _sc_segment_sums(x_all, eidx, zeros_hbm)

    wmats = []
    for d in 'vef':
        rels = DST_RELS[d]
        K = float(len(rels))
        Fd = FEATS[d]
        wmats.append(jnp.zeros((W8, H), jnp.float32)
                     .at[:Fd, :].set(params['Ws_' + d]
                                     + sum(params['Wr_' + r]
                                           for r in rels) / K)
                     .at[Fd, :].set(params['bs_' + d]
                                    + sum(params['bl_' + r]
                                          + params['br_' + r]
                                          for r in rels) / K))
        for r in rels:
            wmats.append(jnp.zeros((W8, H), jnp.float32)
                         .at[:FEATS[r[0]], :].set(params['Wl_' + r] / K))
    weights = jnp.concatenate(wmats, axis=0)

    out_v, out_e, out_f = _tc_epilogue(x_all, slab, weights)
    return (out_v, out_e, out_f)
